# baseline (speedup 1.0000x reference)
import functools

import jax
import jax.numpy as jnp
from jax import lax
from jax.experimental import pallas as pl
from jax.experimental.pallas import tpu as pltpu
from jax.experimental.pallas import tpu_sc as plsc

_BETA = 0.01
_EPS = 1e-6
_S, _B, _J, _X = 2, 8, 17, 64
_N = _X * _X * _X
_ROWS = _S * _B * _J
_RPB = 8
_NBLK = _ROWS // _RPB
_BLK_PER_STAGE = (_B * _J) // _RPB
_LANES = 128
_TROWS = _ROWS * (_N // _LANES)
_NWORKERS = 32
_PER_W = 16
_PAD = _NWORKERS * _PER_W


def _sc_gather_body(table_hbm, rows_hbm, out_hbm, rowv, rowsv, sem):
    wid = lax.axis_index("s") * 2 + lax.axis_index("c")
    base = wid * _PER_W
    pltpu.sync_copy(rows_hbm.at[pl.ds(base, _PER_W)], rowv)
    pltpu.async_copy(table_hbm.at[rowv], rowsv, sem).wait()
    pltpu.sync_copy(rowsv, out_hbm.at[pl.ds(base, _PER_W)])


@functools.lru_cache(maxsize=1)
def _make_sc_gather():
    return functools.partial(
        pl.kernel,
        mesh=plsc.VectorSubcoreMesh(core_axis_name="c", subcore_axis_name="s"),
        out_type=jax.ShapeDtypeStruct((_PAD, _LANES), jnp.float32),
        scratch_types=[
            pltpu.VMEM((_PER_W,), jnp.int32),
            pltpu.VMEM((_PER_W, _LANES), jnp.float32),
            pltpu.SemaphoreType.DMA,
        ],
    )(_sc_gather_body)


_CH = _N // _LANES
_NBUF = 4


def _tc_loss_body(lane_ref, grow_ref, x_hbm, out_ref, buf, sems):
    i = pl.program_id(0)

    def chunk(tt, c):
        slot = lax.rem(tt, _NBUF)
        return pltpu.make_async_copy(
            x_hbm.at[pl.ds(tt * _RPB + c, 1)],
            buf.at[slot, pl.ds(c, 1)],
            sems.at[slot, c])

    @pl.when(i == 0)
    def _prologue():
        out_ref[0] = 0.0
        out_ref[1] = 0.0
        for k in range(_NBUF):
            for c in range(_RPB):
                chunk(jnp.int32(k), c).start()

    for c in range(_RPB):
        chunk(i, c).wait()
    x = buf[lax.rem(i, _NBUF)]
    m = jnp.max(x, axis=(1, 2))
    s = jnp.sum(jnp.exp(x - m[:, None, None]), axis=(1, 2))
    lse = m + jnp.log(s)

    @pl.when(i + _NBUF < _NBLK)
    def _issue_next():
        for c in range(_RPB):
            chunk(i + _NBUF, c).start()

    rows = grow_ref[0]
    lane = lane_ref[0, 0, :]
    col = lax.broadcasted_iota(jnp.int32, (_RPB, _LANES), 1)
    g = jnp.sum(jnp.where(col == lane[:, None], rows, 0.0), axis=1)
    term = -jnp.log(jnp.exp(g - lse) + _EPS)
    partial = jnp.sum(term)
    in_stage0 = i < _BLK_PER_STAGE
    out_ref[0] += jnp.where(in_stage0, partial, 0.0)
    out_ref[1] += jnp.where(in_stage0, 0.0, partial)


def _tc_loss(lane3, grows3, x3):
    return pl.pallas_call(
        _tc_loss_body,
        grid=(_NBLK,),
        in_specs=[
            pl.BlockSpec((1, 1, _RPB), lambda i: (i, 0, 0)),
            pl.BlockSpec((1, _RPB, _LANES), lambda i: (i, 0, 0)),
            pl.BlockSpec(memory_space=pl.ANY),
        ],
        out_specs=pl.BlockSpec(memory_space=pltpu.SMEM),
        out_shape=jax.ShapeDtypeStruct((2,), jnp.float32),
        scratch_shapes=[
            pltpu.VMEM((_NBUF, _RPB, _CH, _LANES), jnp.float32),
            pltpu.SemaphoreType.DMA((_NBUF, _RPB)),
        ],
    )(lane3, grows3, x3)


def kernel(volumes_batch_pred_cat, label, vmax_cat, vmin_cat):
    vol = volumes_batch_pred_cat
    vmin = vmin_cat.transpose(1, 0, 2)
    vmax = vmax_cat.transpose(1, 0, 2)
    mean = (vmax + vmin) * 0.5
    scale = (vmax - vmin) * 0.5
    gt = (label[None] - mean[:, :, None, :]) / scale[:, :, None, :]
    idx = jnp.floor((gt + 1.0) * 0.5 * (_X - 1)).astype(jnp.int32)
    imax = jnp.max(idx, axis=(1, 2, 3))
    imin = jnp.min(idx, axis=(1, 2, 3))
    in_bounds = (imax < _X) & (imax > 0) & (imin < _X) & (imin > 0)

    idx_c = jnp.clip(idx, 0, _X - 1)
    fi = (idx_c[..., 0] * (_X * _X) + idx_c[..., 1] * _X
          + idx_c[..., 2]).reshape(_ROWS).astype(jnp.int32)
    r = jnp.arange(_ROWS, dtype=jnp.int32)
    trow = r * (_N // _LANES) + fi // _LANES
    lane = fi % _LANES
    trow_p = jnp.zeros((_PAD,), jnp.int32).at[:_ROWS].set(trow)

    table = vol.reshape(_TROWS, _LANES)
    grows3 = _make_sc_gather()(table, trow_p)[:_ROWS].reshape(
        _NBLK, _RPB, _LANES)

    x3 = vol.reshape(_ROWS, _CH, _LANES)
    lane3 = lane.reshape(_NBLK, 1, _RPB)
    sums = _tc_loss(lane3, grows3, x3)

    loss = _BETA * sums / (_B * _J)
    total = (jnp.where(in_bounds[0], loss[0], 0.0)
             + jnp.where(in_bounds[1], loss[1], 0.0))
    return total.astype(jnp.float32)


# speedup vs baseline: 4.9847x; 4.9847x over previous
import jax
import jax.numpy as jnp
from jax import lax
from jax.experimental import pallas as pl
from jax.experimental.pallas import tpu as pltpu

_BETA = 0.01
_EPS = 1e-6
_S, _B, _J, _X = 2, 8, 17, 64
_ROWS = _S * _B * _J
_RPB = 8
_NBLK = _ROWS // _RPB
_BLK_PER_STAGE = (_B * _J) // _RPB
_ZL = _X
_CH = _X * _X
_NBUF = 2


def _tc_loss_body(xy_ref, lz_ref, x_hbm, out_ref, buf, sems):
    i = pl.program_id(0)

    def chunk(tt, c):
        slot = lax.rem(tt, _NBUF)
        return pltpu.make_async_copy(
            x_hbm.at[pl.ds(tt * _RPB + c, 1)],
            buf.at[slot, pl.ds(c, 1)],
            sems.at[slot, c])

    @pl.when(i == 0)
    def _prologue():
        out_ref[0] = 0.0
        out_ref[1] = 0.0
        for k in range(_NBUF):
            for c in range(_RPB):
                chunk(jnp.int32(k), c).start()

    for c in range(_RPB):
        chunk(i, c).wait()
    slot = lax.rem(i, _NBUF)
    x = buf[slot]
    m = jnp.max(x, axis=(1, 2))
    s = jnp.sum(jnp.exp(x - m[:, None, None]), axis=(1, 2))
    lse = m + jnp.log(s)

    @pl.when(i + _NBUF < _NBLK)
    def _issue_next():
        for c in range(_RPB):
            chunk(i + _NBUF, c).start()

    gs = []
    sub_i = lax.broadcasted_iota(jnp.int32, (8, _ZL), 0)
    lan_i = lax.broadcasted_iota(jnp.int32, (8, _ZL), 1)
    for r in range(_RPB):
        xy = xy_ref[0, 0, r]
        lz = lz_ref[0, 0, r]
        base = (xy // 8) * 8
        slab = buf[slot, r, pl.ds(base, 8), :]
        sel = (sub_i == xy % 8) & (lan_i == lz)
        gs.append(jnp.sum(jnp.where(sel, slab, 0.0)))
    g = jnp.stack(gs)
    term = -jnp.log(jnp.exp(g - lse) + _EPS)
    partial = jnp.sum(term)
    in_stage0 = i < _BLK_PER_STAGE
    out_ref[0] += jnp.where(in_stage0, partial, 0.0)
    out_ref[1] += jnp.where(in_stage0, 0.0, partial)


def _tc_loss(xy3, lz3, x3):
    return pl.pallas_call(
        _tc_loss_body,
        grid=(_NBLK,),
        in_specs=[
            pl.BlockSpec((1, 1, _RPB), lambda i: (i, 0, 0),
                         memory_space=pltpu.SMEM),
            pl.BlockSpec((1, 1, _RPB), lambda i: (i, 0, 0),
                         memory_space=pltpu.SMEM),
            pl.BlockSpec(memory_space=pl.ANY),
        ],
        out_specs=pl.BlockSpec(memory_space=pltpu.SMEM),
        out_shape=jax.ShapeDtypeStruct((2,), jnp.float32),
        scratch_shapes=[
            pltpu.VMEM((_NBUF, _RPB, _CH, _ZL), jnp.float32),
            pltpu.SemaphoreType.DMA((_NBUF, _RPB)),
        ],
    )(xy3, lz3, x3)


def kernel(volumes_batch_pred_cat, label, vmax_cat, vmin_cat):
    vol = volumes_batch_pred_cat
    vmin = vmin_cat.transpose(1, 0, 2)
    vmax = vmax_cat.transpose(1, 0, 2)
    mean = (vmax + vmin) * 0.5
    scale = (vmax - vmin) * 0.5
    gt = (label[None] - mean[:, :, None, :]) / scale[:, :, None, :]
    idx = jnp.floor((gt + 1.0) * 0.5 * (_X - 1)).astype(jnp.int32)
    imax = jnp.max(idx, axis=(1, 2, 3))
    imin = jnp.min(idx, axis=(1, 2, 3))
    in_bounds = (imax < _X) & (imax > 0) & (imin < _X) & (imin > 0)

    idx_c = jnp.clip(idx, 0, _X - 1)
    xy = (idx_c[..., 0] * _X + idx_c[..., 1]).reshape(_ROWS).astype(jnp.int32)
    lz = idx_c[..., 2].reshape(_ROWS).astype(jnp.int32)

    x3 = vol.reshape(_ROWS, _CH, _ZL)
    xy3 = xy.reshape(_NBLK, 1, _RPB)
    lz3 = lz.reshape(_NBLK, 1, _RPB)
    sums = _tc_loss(xy3, lz3, x3)

    loss = _BETA * sums / (_B * _J)
    total = (jnp.where(in_bounds[0], loss[0], 0.0)
             + jnp.where(in_bounds[1], loss[1], 0.0))
    return total.astype(jnp.float32)


# speedup vs baseline: 4.9950x; 1.0021x over previous
import jax
import jax.numpy as jnp
from jax import lax
from jax.experimental import pallas as pl
from jax.experimental.pallas import tpu as pltpu

_BETA = 0.01
_EPS = 1e-6
_S, _B, _J, _X = 2, 8, 17, 64
_ROWS = _S * _B * _J
_RPB = 8
_NBLK = _ROWS // _RPB
_BLK_PER_STAGE = (_B * _J) // _RPB
_ZL = _X
_CH = _X * _X
_NBUF = 3


def _tc_loss_body(xy_ref, lz_ref, x_hbm, out_ref, buf, sems):
    i = pl.program_id(0)

    def chunk(tt, c):
        slot = lax.rem(tt, _NBUF)
        return pltpu.make_async_copy(
            x_hbm.at[pl.ds(tt * _RPB + c, 1)],
            buf.at[slot, pl.ds(c, 1)],
            sems.at[slot, c])

    @pl.when(i == 0)
    def _prologue():
        out_ref[0] = 0.0
        out_ref[1] = 0.0
        for k in range(_NBUF):
            for c in range(_RPB):
                chunk(jnp.int32(k), c).start()

    for c in range(_RPB):
        chunk(i, c).wait()
    slot = lax.rem(i, _NBUF)
    x = buf[slot]
    m = jnp.max(x, axis=(1, 2))
    s = jnp.sum(jnp.exp(x - m[:, None, None]), axis=(1, 2))
    lse = m + jnp.log(s)

    @pl.when(i + _NBUF < _NBLK)
    def _issue_next():
        for c in range(_RPB):
            chunk(i + _NBUF, c).start()

    gs = []
    sub_i = lax.broadcasted_iota(jnp.int32, (8, _ZL), 0)
    lan_i = lax.broadcasted_iota(jnp.int32, (8, _ZL), 1)
    for r in range(_RPB):
        xy = xy_ref[0, 0, r]
        lz = lz_ref[0, 0, r]
        base = (xy // 8) * 8
        slab = buf[slot, r, pl.ds(base, 8), :]
        sel = (sub_i == xy % 8) & (lan_i == lz)
        gs.append(jnp.sum(jnp.where(sel, slab, 0.0)))
    g = jnp.stack(gs)
    term = -jnp.log(jnp.exp(g - lse) + _EPS)
    partial = jnp.sum(term)
    in_stage0 = i < _BLK_PER_STAGE
    out_ref[0] += jnp.where(in_stage0, partial, 0.0)
    out_ref[1] += jnp.where(in_stage0, 0.0, partial)


def _tc_loss(xy3, lz3, x3):
    return pl.pallas_call(
        _tc_loss_body,
        grid=(_NBLK,),
        in_specs=[
            pl.BlockSpec((1, 1, _RPB), lambda i: (i, 0, 0),
                         memory_space=pltpu.SMEM),
            pl.BlockSpec((1, 1, _RPB), lambda i: (i, 0, 0),
                         memory_space=pltpu.SMEM),
            pl.BlockSpec(memory_space=pl.ANY),
        ],
        out_specs=pl.BlockSpec(memory_space=pltpu.SMEM),
        out_shape=jax.ShapeDtypeStruct((2,), jnp.float32),
        scratch_shapes=[
            pltpu.VMEM((_NBUF, _RPB, _CH, _ZL), jnp.float32),
            pltpu.SemaphoreType.DMA((_NBUF, _RPB)),
        ],
    )(xy3, lz3, x3)


def kernel(volumes_batch_pred_cat, label, vmax_cat, vmin_cat):
    vol = volumes_batch_pred_cat
    vmin = vmin_cat.transpose(1, 0, 2)
    vmax = vmax_cat.transpose(1, 0, 2)
    mean = (vmax + vmin) * 0.5
    scale = (vmax - vmin) * 0.5
    gt = (label[None] - mean[:, :, None, :]) / scale[:, :, None, :]
    idx = jnp.floor((gt + 1.0) * 0.5 * (_X - 1)).astype(jnp.int32)
    imax = jnp.max(idx, axis=(1, 2, 3))
    imin = jnp.min(idx, axis=(1, 2, 3))
    in_bounds = (imax < _X) & (imax > 0) & (imin < _X) & (imin > 0)

    idx_c = jnp.clip(idx, 0, _X - 1)
    xy = (idx_c[..., 0] * _X + idx_c[..., 1]).reshape(_ROWS).astype(jnp.int32)
    lz = idx_c[..., 2].reshape(_ROWS).astype(jnp.int32)

    x3 = vol.reshape(_ROWS, _CH, _ZL)
    xy3 = xy.reshape(_NBLK, 1, _RPB)
    lz3 = lz.reshape(_NBLK, 1, _RPB)
    sums = _tc_loss(xy3, lz3, x3)

    loss = _BETA * sums / (_B * _J)
    total = (jnp.where(in_bounds[0], loss[0], 0.0)
             + jnp.where(in_bounds[1], loss[1], 0.0))
    return total.astype(jnp.float32)
